# Initial kernel scaffold; baseline (speedup 1.0000x reference)
import jax
import jax.numpy as jnp
from jax.experimental import pallas as pl


def kernel(src_feas, src_labels, tgt_feas, tgt_logits):
    raise NotImplementedError("write your pallas kernel here")


# speedup vs baseline: 1.0576x; 1.0576x over previous
import jax
import jax.numpy as jnp
from jax import lax
from jax.experimental import pallas as pl
from jax.experimental.pallas import tpu as pltpu

B, D, C = 512, 32, 64
ROWS = 128
STEPS = B // ROWS
NEG_INF = float("-inf")

_NT = (((1,), (1,)), ((), ()))


def _nce_body(src_ref, lab_ref, tgt_ref, logit_ref, out_ref,
              cc_oh_ref, w_ref, acc_ref):
    i = pl.program_id(0)

    @pl.when(i == 0)
    def _mine():
        logits = logit_ref[:]
        col = lax.broadcasted_iota(jnp.int32, (B, C), 1)
        rmax = jnp.max(logits, axis=1, keepdims=True)
        cc = jnp.min(jnp.where(logits == rmax, col, C), axis=1,
                     keepdims=True)
        cc_oh = (col == cc).astype(jnp.float32)
        cc_oh_ref[:, :] = cc_oh
        lab_oh = (lab_ref[:] == col).astype(jnp.float32)
        hist = jnp.sum(lab_oh, axis=0, keepdims=True)
        w = lax.dot_general(hist, cc_oh, _NT,
                            preferred_element_type=jnp.float32)
        w_ref[:, :] = w
        acc_ref[0] = 0.0
        acc_ref[1] = 0.0
        acc_ref[2] = jnp.sum(w)

    src = src_ref[pl.ds(i * ROWS, ROWS), :]
    lab = lab_ref[pl.ds(i * ROWS, ROWS), :]
    s = lax.dot_general(src, tgt_ref[:], _NT,
                        preferred_element_type=jnp.float32)

    colt = lax.broadcasted_iota(jnp.int32, (ROWS, C), 1)
    lab_oh = (lab == colt).astype(jnp.float32)
    m = lax.dot_general(lab_oh, cc_oh_ref[:, :], _NT,
                        preferred_element_type=jnp.float32)
    rowcnt = jnp.sum(m, axis=1, keepdims=True)

    w = w_ref[:, :]
    valid = w > 0.0
    smax = jnp.max(jnp.where(valid, s, NEG_INF), axis=1, keepdims=True)
    e = jnp.where(valid, jnp.exp(s - smax), 0.0)
    sumexp = jnp.sum(w * e, axis=1, keepdims=True)
    lse = smax + jnp.log(sumexp)

    acc_ref[0] += jnp.sum(m * s)
    acc_ref[1] += jnp.sum(jnp.where(rowcnt > 0.0, rowcnt * lse, 0.0))

    @pl.when(i == STEPS - 1)
    def _finish():
        nce = (acc_ref[0] - acc_ref[1]) / (-1.0 * B * acc_ref[2])
        out_ref[:, :] = jnp.reshape(nce, (1, 1))


@jax.jit
def kernel(src_feas, src_labels, tgt_feas, tgt_logits):
    labels = src_labels.astype(jnp.int32).reshape(B, 1)
    out = pl.pallas_call(
        _nce_body,
        grid=(STEPS,),
        in_specs=[
            pl.BlockSpec((B, D), lambda i: (0, 0)),
            pl.BlockSpec((B, 1), lambda i: (0, 0)),
            pl.BlockSpec((B, D), lambda i: (0, 0)),
            pl.BlockSpec((B, C), lambda i: (0, 0)),
        ],
        out_specs=pl.BlockSpec((1, 1), lambda i: (0, 0)),
        out_shape=jax.ShapeDtypeStruct((1, 1), jnp.float32),
        scratch_shapes=[
            pltpu.VMEM((B, C), jnp.float32),
            pltpu.VMEM((1, B), jnp.float32),
            pltpu.SMEM((4,), jnp.float32),
        ],
    )(src_feas, labels, tgt_feas, tgt_logits)
    return out[0, 0]
